# baseline (speedup 1.0000x reference)
import jax
import jax.numpy as jnp
from jax.experimental import pallas as pl
from jax.experimental.pallas import tpu as pltpu

N = 100
T = 64
F = 128
H = 256
EMB = 64
FLAT_IN = 32
FLAT_OUT = 64
GNN_HID = 256
GNN_OUT = 128
E = 800
NP = 128
CH = 8
NC = T // CH
GNN_IN = H + FLAT_OUT + EMB


def _sigmoid(x):
    return jnp.tanh(x * 0.5) * 0.5 + 0.5


def _fused_body(xT_hbm, wih_hbm, whh_hbm, wl1_hbm, wr1_hbm, wl2_hbm, wr2_hbm,
                flat_ref, emb_ref, edge_ref, brz_ref, bihn_ref, bhhn_ref,
                Wf_ref, bf_ref, bl1_ref, bl2_ref, Wo_ref, bo_ref,
                out_ref,
                wih_v, whh_v, wl1_v, wr1_v, wl2_v, wr2_v, x_v, gi_v,
                sem_w, sem_g, sem_x):
    f32 = jnp.float32
    bf16 = jnp.bfloat16

    cp_wih = pltpu.make_async_copy(wih_hbm, wih_v, sem_w.at[0])
    cp_whh = pltpu.make_async_copy(whh_hbm, whh_v, sem_w.at[1])
    cp_wih.start()
    cp_whh.start()

    def x_copy(c, buf):
        return pltpu.make_async_copy(
            xT_hbm.at[pl.ds(c * CH * NP, CH * NP), :],
            x_v.at[buf], sem_x.at[buf])

    x_copy(0, 0).start()
    x_copy(1, 1).start()

    cp_l1 = pltpu.make_async_copy(wl1_hbm, wl1_v, sem_g.at[0])
    cp_r1 = pltpu.make_async_copy(wr1_hbm, wr1_v, sem_g.at[1])
    cp_l2 = pltpu.make_async_copy(wl2_hbm, wl2_v, sem_g.at[2])
    cp_r2 = pltpu.make_async_copy(wr2_hbm, wr2_v, sem_g.at[3])
    cp_l1.start()
    cp_r1.start()
    cp_l2.start()
    cp_r2.start()

    cp_wih.wait()
    cp_whh.wait()

    wih = wih_v[...]
    whh = whh_v[...]
    brz = brz_ref[...]
    bihn = bihn_ref[...]
    bhhn = bhhn_ref[...]

    def chunk(c, h):
        buf = jax.lax.rem(c, 2)
        pltpu.make_async_copy(
            xT_hbm.at[pl.ds(c * CH * NP, CH * NP), :],
            x_v.at[buf], sem_x.at[buf]).wait()

        gi_v[...] = jnp.dot(x_v[buf], wih, preferred_element_type=f32)

        def step(s, h):
            gi = gi_v[pl.ds(s * NP, NP), :]
            gh = jnp.dot(h.astype(bf16), whh, preferred_element_type=f32)
            rz = _sigmoid(gi[:, 0:2 * H] + gh[:, 0:2 * H] + brz)
            r = rz[:, 0:H]
            z = rz[:, H:2 * H]
            n = jnp.tanh(gi[:, 2 * H:3 * H] + bihn
                         + r * (gh[:, 2 * H:3 * H] + bhhn))
            return n + z * (h - n)

        h = jax.lax.fori_loop(0, CH, step, h)

        @pl.when(c + 2 < NC)
        def _():
            x_copy_dyn = pltpu.make_async_copy(
                xT_hbm.at[pl.ds((c + 2) * CH * NP, CH * NP), :],
                x_v.at[buf], sem_x.at[buf])
            x_copy_dyn.start()

        return h

    h = jax.lax.fori_loop(0, NC, chunk, jnp.zeros((NP, H), f32))

    flat_enc = (
        jnp.dot(flat_ref[...], Wf_ref[...], preferred_element_type=f32)
        + bf_ref[...]
    )
    gnn_in = jnp.concatenate([h, flat_enc, emb_ref[...]], axis=1)

    src = edge_ref[0:1, :]
    dst = edge_ref[1:2, :]
    iota = jax.lax.broadcasted_iota(jnp.int32, (NP, E), 0)
    oh_src = (iota == src).astype(bf16)
    oh_dst = (iota == dst).astype(bf16)
    A = jax.lax.dot_general(oh_dst, oh_src,
                            (((1,), (1,)), ((), ())),
                            preferred_element_type=f32)
    cnt = jnp.sum(oh_dst.astype(f32), axis=1, keepdims=True)
    denom = jnp.maximum(cnt, 1.0)
    Ab = A.astype(bf16)

    cp_l1.wait()
    cp_r1.wait()
    cp_l2.wait()
    cp_r2.wait()

    mean1 = (jnp.dot(Ab, gnn_in.astype(bf16), preferred_element_type=f32)
             / denom)
    h1 = jax.nn.relu(
        jnp.dot(mean1.astype(bf16), wl1_v[...], preferred_element_type=f32)
        + bl1_ref[...]
        + jnp.dot(gnn_in.astype(bf16), wr1_v[...], preferred_element_type=f32)
    )
    mean2 = (jnp.dot(Ab, h1.astype(bf16), preferred_element_type=f32)
             / denom)
    g2 = (
        jnp.dot(mean2.astype(bf16), wl2_v[...], preferred_element_type=f32)
        + bl2_ref[...]
        + jnp.dot(h1.astype(bf16), wr2_v[...], preferred_element_type=f32)
    )

    cat = jnp.concatenate([g2, h], axis=1)
    logits = jnp.dot(cat, Wo_ref[...], preferred_element_type=f32) + bo_ref[...]
    out_ref[...] = _sigmoid(logits)


def kernel(node_feat, flat, edge_index, W_ih, W_hh, b_ih, b_hh, emb,
           Wf, bf, Wl1, bl1, Wr1, Wl2, bl2, Wr2, Wo, bo):
    f32 = jnp.float32
    bf16 = jnp.bfloat16
    xT = jnp.transpose(node_feat, (1, 0, 2))
    xT = jnp.pad(xT, ((0, 0), (0, NP - N), (0, 0)))
    xT = xT.reshape(T * NP, F).astype(bf16)
    flat_p = jnp.pad(flat, ((0, NP - N), (0, 0)))
    emb_p = jnp.pad(emb, ((0, NP - N), (0, 0)))
    brz = (b_ih[:2 * H] + b_hh[:2 * H]).reshape(1, -1)
    bihn = b_ih[2 * H:].reshape(1, -1)
    bhhn = b_hh[2 * H:].reshape(1, -1)

    hbm = pl.BlockSpec(memory_space=pltpu.MemorySpace.HBM)
    vmem = pl.BlockSpec(memory_space=pltpu.MemorySpace.VMEM)
    out = pl.pallas_call(
        _fused_body,
        out_shape=jax.ShapeDtypeStruct((NP, 1), f32),
        in_specs=[hbm] * 7 + [vmem] * 12,
        out_specs=vmem,
        scratch_shapes=[
            pltpu.VMEM((F, 3 * H), bf16),
            pltpu.VMEM((H, 3 * H), bf16),
            pltpu.VMEM((GNN_IN, GNN_HID), bf16),
            pltpu.VMEM((GNN_IN, GNN_HID), bf16),
            pltpu.VMEM((GNN_HID, GNN_OUT), bf16),
            pltpu.VMEM((GNN_HID, GNN_OUT), bf16),
            pltpu.VMEM((2, CH * NP, F), bf16),
            pltpu.VMEM((CH * NP, 3 * H), jnp.float32),
            pltpu.SemaphoreType.DMA((2,)),
            pltpu.SemaphoreType.DMA((4,)),
            pltpu.SemaphoreType.DMA((2,)),
        ],
    )(
        xT, W_ih.T.astype(bf16), W_hh.T.astype(bf16),
        Wl1.astype(bf16), Wr1.astype(bf16),
        Wl2.astype(bf16), Wr2.astype(bf16),
        flat_p, emb_p, edge_index,
        brz, bihn, bhhn,
        Wf, bf.reshape(1, -1),
        bl1.reshape(1, -1), bl2.reshape(1, -1),
        Wo, bo.reshape(1, 1),
    )
    return out[:N, 0]


# speedup vs baseline: 1.0924x; 1.0924x over previous
import jax
import jax.numpy as jnp
from jax.experimental import pallas as pl
from jax.experimental.pallas import tpu as pltpu

N = 100
T = 64
F = 128
H = 256
EMB = 64
FLAT_IN = 32
FLAT_OUT = 64
GNN_HID = 256
GNN_OUT = 128
E = 800
NP = 128


def _sigmoid(x):
    return jnp.tanh(x * 0.5) * 0.5 + 0.5


def _fused_body(xT_ref, flat_ref, emb_ref, edge_ref,
                WihT_ref, WhhT_ref, brz_ref, bihn_ref, bhhn_ref,
                Wf_ref, bf_ref, Wl1_ref, bl1_ref, Wr1_ref,
                Wl2_ref, bl2_ref, Wr2_ref, Wo_ref, bo_ref,
                out_ref, gi_ref):
    f32 = jnp.float32
    bf16 = jnp.bfloat16
    WhhT = WhhT_ref[...]
    brz = brz_ref[...]
    bihn = bihn_ref[...]
    bhhn = bhhn_ref[...]

    gi_ref[...] = jnp.dot(xT_ref[...], WihT_ref[...],
                          preferred_element_type=f32)

    HB = NP // 2

    def half(gi, h):
        gh = jnp.dot(h.astype(bf16), WhhT, preferred_element_type=f32)
        rz = _sigmoid(gi[:, 0:2 * H] + gh[:, 0:2 * H] + brz)
        r = rz[:, 0:H]
        z = rz[:, H:2 * H]
        n = jnp.tanh(gi[:, 2 * H:3 * H] + bihn + r * (gh[:, 2 * H:3 * H] + bhhn))
        return n + z * (h - n)

    def step(t, carry):
        hA, hB = carry
        giA = gi_ref[pl.ds(t * NP, HB), :]
        giB = gi_ref[pl.ds(t * NP + HB, HB), :]
        return half(giA, hA), half(giB, hB)

    hA, hB = jax.lax.fori_loop(
        0, T, step,
        (jnp.zeros((HB, H), f32), jnp.zeros((HB, H), f32)))
    h = jnp.concatenate([hA, hB], axis=0)

    flat_enc = (
        jnp.dot(flat_ref[...], Wf_ref[...], preferred_element_type=f32)
        + bf_ref[...]
    )
    gnn_in = jnp.concatenate([h, flat_enc, emb_ref[...]], axis=1)

    src = edge_ref[0:1, :]
    dst = edge_ref[1:2, :]
    iota = jax.lax.broadcasted_iota(jnp.int32, (NP, E), 0)
    oh_src = (iota == src).astype(f32)
    oh_dst = (iota == dst).astype(f32)
    A = jax.lax.dot_general(oh_dst, oh_src,
                            (((1,), (1,)), ((), ())),
                            preferred_element_type=f32)
    cnt = jnp.sum(oh_dst, axis=1, keepdims=True)
    denom = jnp.maximum(cnt, 1.0)

    mean1 = jnp.dot(A, gnn_in, preferred_element_type=f32) / denom
    h1 = jax.nn.relu(
        jnp.dot(mean1, Wl1_ref[...], preferred_element_type=f32)
        + bl1_ref[...]
        + jnp.dot(gnn_in, Wr1_ref[...], preferred_element_type=f32)
    )
    mean2 = jnp.dot(A, h1, preferred_element_type=f32) / denom
    g2 = (
        jnp.dot(mean2, Wl2_ref[...], preferred_element_type=f32)
        + bl2_ref[...]
        + jnp.dot(h1, Wr2_ref[...], preferred_element_type=f32)
    )

    cat = jnp.concatenate([g2, h], axis=1)
    logits = jnp.dot(cat, Wo_ref[...], preferred_element_type=f32) + bo_ref[...]
    out_ref[...] = _sigmoid(logits)


def kernel(node_feat, flat, edge_index, W_ih, W_hh, b_ih, b_hh, emb,
           Wf, bf, Wl1, bl1, Wr1, Wl2, bl2, Wr2, Wo, bo):
    f32 = jnp.float32
    bf16 = jnp.bfloat16
    xT = jnp.transpose(node_feat, (1, 0, 2))
    xT = jnp.pad(xT, ((0, 0), (0, NP - N), (0, 0)))
    xT = xT.reshape(T * NP, F).astype(bf16)
    flat_p = jnp.pad(flat, ((0, NP - N), (0, 0)))
    emb_p = jnp.pad(emb, ((0, NP - N), (0, 0)))
    brz = (b_ih[:2 * H] + b_hh[:2 * H]).reshape(1, -1)
    bihn = b_ih[2 * H:].reshape(1, -1)
    bhhn = b_hh[2 * H:].reshape(1, -1)

    out = pl.pallas_call(
        _fused_body,
        out_shape=jax.ShapeDtypeStruct((NP, 1), f32),
        scratch_shapes=[pltpu.VMEM((T * NP, 3 * H), f32)],
    )(
        xT, flat_p, emb_p, edge_index,
        W_ih.T.astype(bf16), W_hh.T.astype(bf16),
        brz, bihn, bhhn,
        Wf, bf.reshape(1, -1),
        Wl1, bl1.reshape(1, -1), Wr1,
        Wl2, bl2.reshape(1, -1), Wr2,
        Wo, bo.reshape(1, 1),
    )
    return out[:N, 0]


# speedup vs baseline: 1.1722x; 1.0731x over previous
import jax
import jax.numpy as jnp
from jax.experimental import pallas as pl
from jax.experimental.pallas import tpu as pltpu

N = 100
T = 64
F = 128
H = 256
EMB = 64
FLAT_IN = 32
FLAT_OUT = 64
GNN_HID = 256
GNN_OUT = 128
E = 800
NP = 128


def _sigmoid(x):
    return jnp.tanh(x * 0.5) * 0.5 + 0.5


def _fused_body(xT_ref, flat_ref, emb_ref, edge_ref,
                WihT_ref, WhhT_ref, brz_ref, bihn_ref, bhhn_ref,
                Wf_ref, bf_ref, Wl1_ref, bl1_ref, Wr1_ref,
                Wl2_ref, bl2_ref, Wr2_ref, Wo_ref, bo_ref,
                out_ref, gi_ref):
    f32 = jnp.float32
    bf16 = jnp.bfloat16
    WhhT = WhhT_ref[...]
    brz = brz_ref[...]
    bihn = bihn_ref[...]
    bhhn = bhhn_ref[...]

    gi_ref[...] = jnp.dot(xT_ref[...], WihT_ref[...],
                          preferred_element_type=f32).astype(bf16)

    def substep(t, h):
        gi = gi_ref[pl.ds(t * NP, NP), :].astype(f32)
        gh = jnp.dot(h.astype(bf16), WhhT, preferred_element_type=f32)
        rz = _sigmoid(gi[:, 0:2 * H] + gh[:, 0:2 * H] + brz)
        r = rz[:, 0:H]
        z = rz[:, H:2 * H]
        n = jnp.tanh(gi[:, 2 * H:3 * H] + bihn + r * (gh[:, 2 * H:3 * H] + bhhn))
        return n + z * (h - n)

    def step(i, h):
        h = substep(2 * i, h)
        return substep(2 * i + 1, h)

    h = jax.lax.fori_loop(0, T // 2, step, jnp.zeros((NP, H), f32))

    flat_enc = (
        jnp.dot(flat_ref[...], Wf_ref[...], preferred_element_type=f32)
        + bf_ref[...]
    )
    gnn_in = jnp.concatenate([h, flat_enc, emb_ref[...]], axis=1)

    src = edge_ref[0:1, :]
    dst = edge_ref[1:2, :]
    iota = jax.lax.broadcasted_iota(jnp.int32, (NP, E), 0)
    oh_src = (iota == src).astype(f32)
    oh_dst = (iota == dst).astype(f32)
    A = jax.lax.dot_general(oh_dst, oh_src,
                            (((1,), (1,)), ((), ())),
                            preferred_element_type=f32)
    cnt = jnp.sum(oh_dst, axis=1, keepdims=True)
    denom = jnp.maximum(cnt, 1.0)

    mean1 = jnp.dot(A, gnn_in, preferred_element_type=f32) / denom
    h1 = jax.nn.relu(
        jnp.dot(mean1, Wl1_ref[...], preferred_element_type=f32)
        + bl1_ref[...]
        + jnp.dot(gnn_in, Wr1_ref[...], preferred_element_type=f32)
    )
    mean2 = jnp.dot(A, h1, preferred_element_type=f32) / denom
    g2 = (
        jnp.dot(mean2, Wl2_ref[...], preferred_element_type=f32)
        + bl2_ref[...]
        + jnp.dot(h1, Wr2_ref[...], preferred_element_type=f32)
    )

    cat = jnp.concatenate([g2, h], axis=1)
    logits = jnp.dot(cat, Wo_ref[...], preferred_element_type=f32) + bo_ref[...]
    out_ref[...] = _sigmoid(logits)


def kernel(node_feat, flat, edge_index, W_ih, W_hh, b_ih, b_hh, emb,
           Wf, bf, Wl1, bl1, Wr1, Wl2, bl2, Wr2, Wo, bo):
    f32 = jnp.float32
    bf16 = jnp.bfloat16
    xT = jnp.transpose(node_feat, (1, 0, 2))
    xT = jnp.pad(xT, ((0, 0), (0, NP - N), (0, 0)))
    xT = xT.reshape(T * NP, F).astype(bf16)
    flat_p = jnp.pad(flat, ((0, NP - N), (0, 0)))
    emb_p = jnp.pad(emb, ((0, NP - N), (0, 0)))
    brz = (b_ih[:2 * H] + b_hh[:2 * H]).reshape(1, -1)
    bihn = b_ih[2 * H:].reshape(1, -1)
    bhhn = b_hh[2 * H:].reshape(1, -1)

    out = pl.pallas_call(
        _fused_body,
        out_shape=jax.ShapeDtypeStruct((NP, 1), f32),
        scratch_shapes=[pltpu.VMEM((T * NP, 3 * H), bf16)],
    )(
        xT, flat_p, emb_p, edge_index,
        W_ih.T.astype(bf16), W_hh.T.astype(bf16),
        brz, bihn, bhhn,
        Wf, bf.reshape(1, -1),
        Wl1, bl1.reshape(1, -1), Wr1,
        Wl2, bl2.reshape(1, -1), Wr2,
        Wo, bo.reshape(1, 1),
    )
    return out[:N, 0]
